# baseline (speedup 1.0000x reference)
import jax
import jax.numpy as jnp
from jax.experimental import pallas as pl
from jax.experimental.pallas import tpu as pltpu

_BN = 512


def _tile_kernel(x_ref, c_ref, dist_ref, assign_ref, c2_ref):
    @pl.when(pl.program_id(0) == 0)
    def _():
        c = c_ref[...]
        c2_ref[...] = jnp.sum(c * c, axis=1, keepdims=True)

    x = x_ref[...]
    x2 = jnp.sum(x * x, axis=1)[None, :]
    dots = jax.lax.dot_general(
        c_ref[...], x, (((1,), (1,)), ((), ())),
        preferred_element_type=jnp.float32,
        precision=jax.lax.Precision.HIGHEST)
    dist = (c2_ref[...] - 2.0 * dots) + x2
    dist_ref[...] = dist

    local_min = jnp.min(dist, axis=0, keepdims=True)
    rows = jax.lax.broadcasted_iota(jnp.int32, dist.shape, 0)
    big = jnp.int32(jnp.iinfo(jnp.int32).max)
    assign_ref[...] = jnp.min(
        jnp.where(dist == local_min, rows, big), axis=0, keepdims=True)


def kernel(inputs, centroids):
    n, d = inputs.shape
    k, _ = centroids.shape
    bn = _BN
    dist, assign = pl.pallas_call(
        _tile_kernel,
        grid=(n // bn,),
        in_specs=[
            pl.BlockSpec((bn, d), lambda j: (j, 0)),
            pl.BlockSpec((k, d), lambda j: (0, 0)),
        ],
        out_specs=[
            pl.BlockSpec((k, bn), lambda j: (0, j)),
            pl.BlockSpec((1, bn), lambda j: (0, j)),
        ],
        out_shape=[
            jax.ShapeDtypeStruct((k, n), jnp.float32),
            jax.ShapeDtypeStruct((1, n), jnp.int32),
        ],
        scratch_shapes=[
            pltpu.VMEM((k, 1), jnp.float32),
        ],
        compiler_params=pltpu.CompilerParams(
            dimension_semantics=("arbitrary",)),
    )(inputs, centroids)
    return dist, assign[0]


# speedup vs baseline: 1.3560x; 1.3560x over previous
import jax
import jax.numpy as jnp
from jax.experimental import pallas as pl
from jax.experimental.pallas import tpu as pltpu

_BN = 512


def _tile_kernel(x_ref, c_ref, dist_ref, assign_ref, c2_ref, ch_ref, cl_ref):
    @pl.when(pl.program_id(0) == 0)
    def _():
        c = c_ref[...]
        c2_ref[...] = jnp.sum(c * c, axis=1, keepdims=True)
        ch = c.astype(jnp.bfloat16)
        ch_ref[...] = ch
        cl_ref[...] = (c - ch.astype(jnp.float32)).astype(jnp.bfloat16)

    x = x_ref[...]
    x2 = jnp.sum(x * x, axis=1)[None, :]
    xh = x.astype(jnp.bfloat16)
    xl = (x - xh.astype(jnp.float32)).astype(jnp.bfloat16)

    dims = (((1,), (1,)), ((), ()))
    f32 = jnp.float32
    dots = jax.lax.dot_general(ch_ref[...], xh, dims, preferred_element_type=f32)
    dots += jax.lax.dot_general(ch_ref[...], xl, dims, preferred_element_type=f32)
    dots += jax.lax.dot_general(cl_ref[...], xh, dims, preferred_element_type=f32)

    dist = (c2_ref[...] - 2.0 * dots) + x2
    dist_ref[...] = dist

    local_min = jnp.min(dist, axis=0, keepdims=True)
    rows = jax.lax.broadcasted_iota(jnp.int32, dist.shape, 0)
    big = jnp.int32(jnp.iinfo(jnp.int32).max)
    assign_ref[...] = jnp.min(
        jnp.where(dist == local_min, rows, big), axis=0, keepdims=True)


def kernel(inputs, centroids):
    n, d = inputs.shape
    k, _ = centroids.shape
    bn = _BN
    dist, assign = pl.pallas_call(
        _tile_kernel,
        grid=(n // bn,),
        in_specs=[
            pl.BlockSpec((bn, d), lambda j: (j, 0)),
            pl.BlockSpec((k, d), lambda j: (0, 0)),
        ],
        out_specs=[
            pl.BlockSpec((k, bn), lambda j: (0, j)),
            pl.BlockSpec((1, bn), lambda j: (0, j)),
        ],
        out_shape=[
            jax.ShapeDtypeStruct((k, n), jnp.float32),
            jax.ShapeDtypeStruct((1, n), jnp.int32),
        ],
        scratch_shapes=[
            pltpu.VMEM((k, 1), jnp.float32),
            pltpu.VMEM((k, d), jnp.bfloat16),
            pltpu.VMEM((k, d), jnp.bfloat16),
        ],
        compiler_params=pltpu.CompilerParams(
            dimension_semantics=("arbitrary",)),
    )(inputs, centroids)
    return dist, assign[0]
